# baseline (speedup 1.0000x reference)
import functools

import jax
import jax.numpy as jnp
from jax import lax
from jax.experimental import pallas as pl
from jax.experimental.pallas import tpu as pltpu
from jax.experimental.pallas import tpu_sc as plsc

N, E, LE, PE = 50000, 800000, 1200000, 400000
IN_F, H, S, ED = 128, 64, 128, 16
NPAIRS, B = 2048, 1024
N_ITER = 3

BNR = 2000
BE = 8000
BP = 8000
BJ = 4000

_f32 = jnp.float32


def _full(shape):
    return pl.BlockSpec(shape, lambda *a: tuple(0 for _ in shape))


def _rows(bshape):
    return pl.BlockSpec(bshape, lambda i: (i,) + tuple(0 for _ in bshape[1:]))


def _stat_rows(t):
    return jnp.concatenate(
        [t.sum(0, keepdims=True), (t * t).sum(0, keepdims=True),
         jnp.zeros((6, t.shape[1]), _f32)], axis=0)


def _bn_coeff(st, n, g, b, eps=1e-5):
    m = st[0] / n
    v = st[1] / n - m * m
    inv = g / jnp.sqrt(v + eps)
    return inv, b - m * inv


NW = 32
SEG_PAD = 1280
_SC_PARAMS = pltpu.CompilerParams(use_tc_tiling_on_sc=False,
                                  needs_layout_passes=False)


def _sc_multigather(tables, idxs, share_prev, K):
    n = len(tables)
    Bn = idxs[0].shape[0]
    D = tables[0].shape[1]
    C = -(-Bn // K)
    per_w = -(-C // NW)
    mesh = plsc.VectorSubcoreMesh(core_axis_name="c", subcore_axis_name="s")
    out_type = [jax.ShapeDtypeStruct((Bn, D), _f32) for _ in range(n)]
    scratch = [pltpu.VMEM((K,), jnp.int32), pltpu.VMEM((K, D), _f32),
               pltpu.SemaphoreType.DMA]

    def body(*refs):
        tabs = refs[:n]
        idr = refs[n:2 * n]
        outs = refs[2 * n:3 * n]
        idx_v, rows_v, sem = refs[3 * n:]
        wid = lax.axis_index("s") * 2 + lax.axis_index("c")

        def chunk(j, carry):
            c = wid + j * NW
            s = pl.multiple_of(jnp.minimum(c * K, Bn - K), 8)

            @pl.when(c < C)
            def _():
                for t in range(n):
                    if not share_prev[t]:
                        pltpu.sync_copy(idr[t].at[pl.ds(s, K)], idx_v)
                    pltpu.async_copy(tabs[t].at[idx_v], rows_v, sem).wait()
                    pltpu.sync_copy(rows_v, outs[t].at[pl.ds(s, K)])

            return carry

        lax.fori_loop(0, per_w, chunk, 0)

    return pl.kernel(body, out_type=out_type, mesh=mesh,
                     scratch_types=scratch,
                     compiler_params=_SC_PARAMS)(*tables, *idxs)


def _sc_segsum(table, idxp, segp, starts, ends, zeros, nchunk, CH, K=512):
    ZR = CH // 16
    npad = nchunk * CH
    DUM = CH
    mesh = plsc.VectorSubcoreMesh(core_axis_name="c", subcore_axis_name="s")
    out_type = jax.ShapeDtypeStruct((npad, H), _f32)
    scratch = [pltpu.VMEM((K,), jnp.int32), pltpu.VMEM((K,), jnp.int32),
               pltpu.VMEM((K,), jnp.int32), pltpu.VMEM((K,), jnp.int32),
               pltpu.VMEM((K, H), _f32), pltpu.VMEM((ZR, H), _f32),
               pltpu.VMEM((16,), jnp.int32), pltpu.VMEM((16,), jnp.int32),
               pltpu.VMEM_SHARED((CH + 8, H), _f32),
               pltpu.SemaphoreType.DMA]

    def body(tab, idr, sgr, str_r, end_r, zr, out, raw0, raw1, i0_v, li_v,
             rows_v, zero_v, sv, ev, shared, sem):
        cid = lax.axis_index("c")
        sid = lax.axis_index("s")
        iota = lax.iota(jnp.int32, 16)
        pltpu.sync_copy(zr, zero_v)

        def chunk(j, carry):
            c = cid + 2 * j
            base = c * CH
            c8 = pl.multiple_of((c // 8) * 8, 8)
            pltpu.sync_copy(str_r.at[pl.ds(c8, 8)], sv.at[pl.ds(0, 8)])
            pltpu.sync_copy(end_r.at[pl.ds(c8, 8)], ev.at[pl.ds(0, 8)])
            lane = c - c8
            e0 = jnp.sum(jnp.where(iota == lane, sv[...], 0))
            e1 = jnp.sum(jnp.where(iota == lane, ev[...], 0))
            pltpu.sync_copy(zero_v, shared.at[pl.ds(sid * ZR, ZR)])

            @pl.when(sid == 0)
            def _():
                pltpu.sync_copy(zero_v.at[pl.ds(0, 8)],
                                shared.at[pl.ds(CH, 8)])

            plsc.subcore_barrier()
            cnt = e1 - e0
            L8 = (((cnt + 15) // 16 + 7) // 8) * 8
            s0 = e0 + sid * L8
            et = jnp.minimum(s0 + L8, e1)
            nb = (L8 + K - 1) // K

            def blk(b, cc):
                s = pl.multiple_of(s0 + b * K, 8)
                pltpu.sync_copy(idr.at[pl.ds(s, K)], raw0)
                pltpu.sync_copy(sgr.at[pl.ds(s, K)], raw1)
                rem = et - s
                for k in range(K // 16):
                    sl = pl.ds(k * 16, 16)
                    local = raw1[sl] - base
                    m = ((iota + (k * 16) < rem) & (local >= 0)
                         & (local < CH))
                    i0_v[sl] = jnp.where(m, raw0[sl], 0)
                    li_v[sl] = jnp.where(m, local, DUM)
                pltpu.async_copy(tab.at[i0_v], rows_v, sem).wait()
                pltpu.sync_copy(rows_v, shared.at[li_v], add=True)
                return cc

            lax.fori_loop(0, nb, blk, 0)
            plsc.subcore_barrier()
            pltpu.sync_copy(shared.at[pl.ds(sid * ZR, ZR)],
                            out.at[pl.ds(pl.multiple_of(base + sid * ZR, 8),
                                         ZR)])
            plsc.subcore_barrier()
            return carry

        lax.fori_loop(0, nchunk // 2, chunk, 0)

    return pl.kernel(body, out_type=out_type, mesh=mesh,
                     scratch_types=scratch,
                     compiler_params=_SC_PARAMS)(table, idxp, segp, starts,
                                                 ends, zeros)


def _pad_entries(idx, seg):
    pad0 = jnp.zeros((SEG_PAD,), jnp.int32)
    padm = jnp.full((SEG_PAD,), -1, jnp.int32)
    return (jnp.concatenate([idx, pad0]), jnp.concatenate([seg, padm]))


def _chunk_bounds(seg_sorted, nchunk, CH):
    cb = jnp.searchsorted(
        seg_sorted, jnp.arange(nchunk + 1, dtype=jnp.int32) * CH
    ).astype(jnp.int32)
    starts = (cb[:-1] // 8) * 8
    ends = (cb[1:] // 8) * 8 + 8
    lpad = (nchunk // 8) * 8 + 16
    z = jnp.zeros((lpad - nchunk,), jnp.int32)
    return jnp.concatenate([starts, z]), jnp.concatenate([ends, z])


def _mlpa_body(x_ref, w1_ref, b1_ref, p1_ref, w2_ref, b2_ref, t1_ref, st_ref):
    i = pl.program_id(0)
    h = x_ref[...] @ w1_ref[...] + b1_ref[...]
    h = jnp.where(h >= 0, h, p1_ref[0, 0] * h)
    t = h @ w2_ref[...] + b2_ref[...]
    t1_ref[...] = t

    @pl.when(i == 0)
    def _():
        st_ref[...] = jnp.zeros_like(st_ref)

    st_ref[...] += _stat_rows(t)


def _mlpa(x, w1, b1, p1, w2, b2):
    return pl.pallas_call(
        _mlpa_body,
        grid=(N // BNR,),
        in_specs=[_rows((BNR, IN_F)), _full((IN_F, H)), _full((1, H)),
                  _full((1, 1)), _full((H, H)), _full((1, H))],
        out_specs=[_rows((BNR, H)), _full((8, H))],
        out_shape=[jax.ShapeDtypeStruct((N, H), _f32),
                   jax.ShapeDtypeStruct((8, H), _f32)],
    )(x, w1, b1.reshape(1, H), p1.reshape(1, 1), w2, b2.reshape(1, H))


def _mlpb_body(t1_ref, sc_ref, sh_ref, p2_ref, w3_ref, b3_ref, t2_ref, st_ref):
    i = pl.program_id(0)
    u = t1_ref[...] * sc_ref[...] + sh_ref[...]
    u = jnp.where(u >= 0, u, p2_ref[0, 0] * u)
    t = u @ w3_ref[...] + b3_ref[...]
    t2_ref[...] = t

    @pl.when(i == 0)
    def _():
        st_ref[...] = jnp.zeros_like(st_ref)

    st_ref[...] += _stat_rows(t)


def _mlpb(t1, sc, sh, p2, w3, b3):
    return pl.pallas_call(
        _mlpb_body,
        grid=(N // BNR,),
        in_specs=[_rows((BNR, H)), _full((1, H)), _full((1, H)),
                  _full((1, 1)), _full((H, H)), _full((1, H))],
        out_specs=[_rows((BNR, H)), _full((8, H))],
        out_shape=[jax.ShapeDtypeStruct((N, H), _f32),
                   jax.ShapeDtypeStruct((8, H), _f32)],
    )(t1, sc.reshape(1, H), sh.reshape(1, H), p2.reshape(1, 1), w3,
      b3.reshape(1, H))


def _mlpc_body(t2_ref, sc_ref, sh_ref, wi_ref, wj_ref, h_ref, hi_ref, hj_ref):
    hh = t2_ref[...] * sc_ref[...] + sh_ref[...]
    h_ref[...] = hh
    hi_ref[...] = hh @ wi_ref[...]
    hj_ref[...] = hh @ wj_ref[...]


def _mlpc(t2, sc, sh, wi, wj):
    return pl.pallas_call(
        _mlpc_body,
        grid=(N // BNR,),
        in_specs=[_rows((BNR, H)), _full((1, H)), _full((1, H)),
                  _full((H, H)), _full((H, H))],
        out_specs=[_rows((BNR, H))] * 3,
        out_shape=[jax.ShapeDtypeStruct((N, H), _f32)] * 3,
    )(t2, sc.reshape(1, H), sh.reshape(1, H), wi, wj)


def _edge_body(hid_ref, hjs_ref, hs_ref, ef_ref, bb_ref, sp_ref, sw_ref,
               sb_ref, eew_ref, eeb_ref, ea_ref, ew_ref):
    a = hid_ref[...] + hjs_ref[...] + bb_ref[...]
    a = jnp.where(a >= 0, a, sp_ref[0, 0] * a)
    t = a @ sw_ref[...] + sb_ref[...]
    ef = ef_ref[...] @ eew_ref[...] + eeb_ref[...]
    alpha = (t * ef).sum(-1, keepdims=True) * (1.0 / 16.0)
    ew = jax.nn.sigmoid(alpha)
    ew_ref[...] = ew
    ea_ref[...] = hs_ref[...] * ew


def _edge_gate(HiD, HjS, Hs, edge_feats, bb, sp, sw, sb, eew, eeb):
    return pl.pallas_call(
        _edge_body,
        grid=(E // BE,),
        in_specs=[_rows((BE, H)), _rows((BE, H)), _rows((BE, H)),
                  _rows((BE, ED)),
                  _full((1, H)), _full((1, 1)), _full((H, H)), _full((1, H)),
                  _full((ED, H)), _full((1, H))],
        out_specs=[_rows((BE, H)), _rows((BE, 1))],
        out_shape=[jax.ShapeDtypeStruct((E, H), _f32),
                   jax.ShapeDtypeStruct((E, 1), _f32)],
    )(HiD, HjS, Hs, edge_feats, bb.reshape(1, H), sp.reshape(1, 1), sw,
      sb.reshape(1, H), eew, eeb.reshape(1, H))


def _upd_body(ea_ref, agg_ref, ew_ref, out_ref):
    out_ref[...] = ea_ref[...] + agg_ref[...] * ew_ref[...]


def _lg_update(ea, agg, ew):
    return pl.pallas_call(
        _upd_body,
        grid=(E // BE,),
        in_specs=[_rows((BE, H)), _rows((BE, H)), _rows((BE, 1))],
        out_specs=_rows((BE, H)),
        out_shape=jax.ShapeDtypeStruct((E, H), _f32),
    )(ea, agg, ew)


def _f1_body(h_ref, ag_ref, h2_ref, st_ref):
    i = pl.program_id(0)
    t = h_ref[...] + ag_ref[...]
    h2_ref[...] = t

    @pl.when(i == 0)
    def _():
        st_ref[...] = jnp.zeros_like(st_ref)

    st_ref[...] += _stat_rows(t)


def _f1(h, aggh):
    return pl.pallas_call(
        _f1_body,
        grid=(N // BNR,),
        in_specs=[_rows((BNR, H)), _rows((BNR, H))],
        out_specs=[_rows((BNR, H)), _full((8, H))],
        out_shape=[jax.ShapeDtypeStruct((N, H), _f32),
                   jax.ShapeDtypeStruct((8, H), _f32)],
    )(h, aggh)


def _lin_body(zin_ref, sc_ref, sh_ref, p_ref, w_ref, wb_ref, z_ref, st_ref):
    i = pl.program_id(0)
    u = zin_ref[...] * sc_ref[...] + sh_ref[...]
    u = jnp.where(u >= 0, u, p_ref[0, 0] * u)
    t = u @ w_ref[...] + wb_ref[...]
    z_ref[...] = t

    @pl.when(i == 0)
    def _():
        st_ref[...] = jnp.zeros_like(st_ref)

    st_ref[...] += _stat_rows(t)


def _lin(zin, sc, sh, p, w, wb, fin, fout):
    return pl.pallas_call(
        _lin_body,
        grid=(N // BNR,),
        in_specs=[_rows((BNR, fin)), _full((1, fin)), _full((1, fin)),
                  _full((1, 1)), _full((fin, fout)), _full((1, fout))],
        out_specs=[_rows((BNR, fout)), _full((8, fout))],
        out_shape=[jax.ShapeDtypeStruct((N, fout), _f32),
                   jax.ShapeDtypeStruct((8, fout), _f32)],
    )(zin, sc.reshape(1, fin), sh.reshape(1, fin), p.reshape(1, 1), w,
      wb.reshape(1, fout))


def _mix_body(za_ref, zb_ref, z_ref, st_ref):
    i = pl.program_id(0)
    t = (za_ref[...] + zb_ref[...]) * 0.5
    z_ref[...] = t

    @pl.when(i == 0)
    def _():
        st_ref[...] = jnp.zeros_like(st_ref)

    st_ref[...] += _stat_rows(t)


def _mix(za, zb):
    return pl.pallas_call(
        _mix_body,
        grid=(N // BNR,),
        in_specs=[_rows((BNR, S)), _rows((BNR, S))],
        out_specs=[_rows((BNR, S)), _full((8, S))],
        out_shape=[jax.ShapeDtypeStruct((N, S), _f32),
                   jax.ShapeDtypeStruct((8, S), _f32)],
    )(za, zb)


def _pairproj_body(xj_ref, xi_ref, wk_ref, wq_ref, wip_ref, wjp_ref,
                   kj_ref, qi_ref, pi_ref, pj_ref):
    xj = xj_ref[...]
    xi = xi_ref[...]
    kj_ref[...] = xj @ wk_ref[...]
    qi_ref[...] = xi @ wq_ref[...]
    pi_ref[...] = xi @ wip_ref[...]
    pj_ref[...] = xj @ wjp_ref[...]


def _pairproj(xj, xi, wk, wq, wip, wjp):
    nj = xj.shape[0]
    return pl.pallas_call(
        _pairproj_body,
        grid=(nj // BJ,),
        in_specs=[_rows((BJ, S)), _rows((BJ, S))] + [_full((S, H))] * 4,
        out_specs=[_rows((BJ, H))] * 4,
        out_shape=[jax.ShapeDtypeStruct((nj, H), _f32)] * 4,
    )(xj, xi, wk, wq, wip, wjp)


def _pairatt_body(kj_ref, qi_ref, pi_ref, pj_ref, cb_ref, ca_ref, pv_ref):
    t = jnp.tanh(kj_ref[...] + qi_ref[...] + cb_ref[...])
    att = (t * ca_ref[...]).sum(-1, keepdims=True)
    pv_ref[...] = att * (pi_ref[...] * pj_ref[...])


def _pairatt(KjP, QiP, PiP, PjP, cb, ca):
    return pl.pallas_call(
        _pairatt_body,
        grid=(PE // BP,),
        in_specs=[_rows((BP, H))] * 4 + [_full((1, H)), _full((1, H))],
        out_specs=_rows((BP, H)),
        out_shape=jax.ShapeDtypeStruct((PE, H), _f32),
    )(KjP, QiP, PiP, PjP, cb.reshape(1, H), ca.reshape(1, H))


def _score_body(pg_ref, rg_ref, s_ref):
    s_ref[...] = (pg_ref[...] * rg_ref[...]).sum(-1, keepdims=True)


def _score(pair_g, r_g):
    n = pair_g.shape[0]
    return pl.pallas_call(
        _score_body,
        in_specs=[_full((n, H)), _full((n, H))],
        out_specs=_full((n, 1)),
        out_shape=jax.ShapeDtypeStruct((n, 1), _f32),
    )(pair_g, r_g)


def kernel(x, edge_feats, params, edge_index, line_graph_edge_index,
           pair_edge_index, edge_index_batch, rels, drug_pair_indices,
           node_j_for_pairs, node_i_for_pairs):
    p = params
    src, dst = edge_index[0], edge_index[1]

    t1, st1 = _mlpa(x, p['mlp_w1'], p['mlp_b1'], p['mlp_p1'], p['mlp_w2'],
                    p['mlp_b2'])
    sc1, sh1 = _bn_coeff(st1, N, p['mlp_bn1_g'], p['mlp_bn1_b'])
    t2, st2 = _mlpb(t1, sc1, sh1, p['mlp_p2'], p['mlp_w3'], p['mlp_b3'])
    sc2, sh2 = _bn_coeff(st2, N, p['mlp_bn2_g'], p['mlp_bn2_b'])
    h, hi, hj = _mlpc(t2, sc2, sh2, p['w_i'], p['w_j'])

    HiD, HjS, Hs = _sc_multigather([hi, hj, h], [dst, src, src],
                                   [False, False, True], 1000)
    ea, ew = _edge_gate(HiD, HjS, Hs, edge_feats, p['blk_bias'], p['sml_p'],
                        p['sml_w'], p['sml_b'], p['ee_w'], p['ee_b'])

    lg1s, lg0s = lax.sort((line_graph_edge_index[1], line_graph_edge_index[0]),
                          num_keys=1)
    lg0p, lg1p = _pad_entries(lg0s, lg1s)
    lst, len_ = _chunk_bounds(lg1s, 100, 8000)
    zeros_500 = jnp.zeros((500, H), _f32)
    out = ea
    for _ in range(N_ITER):
        agg = _sc_segsum(out, lg0p, lg1p, lst, len_, zeros_500, 100, 8000)
        out = _lg_update(ea, agg, ew)

    dsts, eids = lax.sort((dst, jnp.arange(E, dtype=dst.dtype)), num_keys=1)
    eidp, dstp = _pad_entries(eids, dsts)
    nst, nen = _chunk_bounds(dsts, 8, 8000)
    aggh = _sc_segsum(out, eidp, dstp, nst, nen, zeros_500, 8, 8000)[:N]

    h2, sth = _f1(h, aggh)
    scl1, shl1 = _bn_coeff(sth, N, p['l1_g'], p['l1_b'])
    z, stz = _lin(h2, scl1, shl1, jnp.float32(1.0), p['l1_w'], p['l1_wb'],
                  H, S)
    scz, shz = _bn_coeff(stz, N, p['l2_g'], p['l2_b'])
    z2, st22 = _lin(z, scz, shz, p['l2_p'], p['l2_w'], p['l2_wb'], S, S)
    sc3, sh3 = _bn_coeff(st22, N, p['l3_g'], p['l3_b'])
    z3, _ = _lin(z2, sc3, sh3, p['l3_p'], p['l3_w'], p['l3_wb'], S, S)
    zB, stB = _mix(z3, z)
    sc4, sh4 = _bn_coeff(stB, N, p['l4_g'], p['l4_b'])
    z4, _ = _lin(zB, sc4, sh4, p['l4_p'], p['l4_w'], p['l4_wb'], S, S)
    zC, _ = _mix(z4, zB)

    xj, xi = _sc_multigather([zC, zC], [node_j_for_pairs, node_i_for_pairs],
                             [False, False], 512)
    Kj, Qi, Pi, Pj = _pairproj(xj, xi, p['ca_wk'], p['ca_wq'], p['i_pro'],
                               p['j_pro'])
    pe0, pe1 = pair_edge_index[0], pair_edge_index[1]
    KjP, QiP, PiP, PjP = _sc_multigather(
        [Kj, Qi, Pi, Pj], [pe0, pe1, pe1, pe0],
        [False, False, True, False], 1000)
    pv = _pairatt(KjP, QiP, PiP, PjP, p['ca_bias'], p['ca_a'])
    arp, eibp = _pad_entries(jnp.arange(PE, dtype=jnp.int32),
                             edge_index_batch)
    pst, pen = _chunk_bounds(edge_index_batch, 2, 1024)
    seg = _sc_segsum(pv, arp, eibp, pst, pen, jnp.zeros((64, H), _f32),
                     2, 1024)
    rels_full = jnp.concatenate([rels, rels])
    pair_g, r_g = _sc_multigather([seg, p['rel_embs']],
                                  [drug_pair_indices, rels_full],
                                  [False, False], 64)
    scores = _score(pair_g, r_g)
    return scores[:B], scores[B:].reshape(B, 1, 1)


# speedup vs baseline: 1.8890x; 1.8890x over previous
import functools

import jax
import jax.numpy as jnp
from jax import lax
from jax.experimental import pallas as pl
from jax.experimental.pallas import tpu as pltpu
from jax.experimental.pallas import tpu_sc as plsc

N, E, LE, PE = 50000, 800000, 1200000, 400000
IN_F, H, S, ED = 128, 64, 128, 16
NPAIRS, B = 2048, 1024
N_ITER = 3

BNR = 2000
BE = 8000
BP = 8000
BJ = 4000

_f32 = jnp.float32


def _full(shape):
    return pl.BlockSpec(shape, lambda *a: tuple(0 for _ in shape))


def _rows(bshape):
    return pl.BlockSpec(bshape, lambda i: (i,) + tuple(0 for _ in bshape[1:]))


def _stat_rows(t):
    return jnp.concatenate(
        [t.sum(0, keepdims=True), (t * t).sum(0, keepdims=True),
         jnp.zeros((6, t.shape[1]), _f32)], axis=0)


def _bn_coeff(st, n, g, b, eps=1e-5):
    m = st[0] / n
    v = st[1] / n - m * m
    inv = g / jnp.sqrt(v + eps)
    return inv, b - m * inv


NW = 32
SEG_PAD = 2560
_SC_PARAMS = pltpu.CompilerParams(use_tc_tiling_on_sc=False,
                                  needs_layout_passes=False)


def _sc_multigather(tables, idxs, share_prev, K):
    n = len(tables)
    Bn = idxs[0].shape[0]
    D = tables[0].shape[1]
    C = -(-Bn // K)
    per_w = -(-C // NW)
    mesh = plsc.VectorSubcoreMesh(core_axis_name="c", subcore_axis_name="s")
    out_type = [jax.ShapeDtypeStruct((Bn, D), _f32) for _ in range(n)]
    scratch = [pltpu.VMEM((K,), jnp.int32), pltpu.VMEM((K, D), _f32),
               pltpu.SemaphoreType.DMA]

    def body(*refs):
        tabs = refs[:n]
        idr = refs[n:2 * n]
        outs = refs[2 * n:3 * n]
        idx_v, rows_v, sem = refs[3 * n:]
        wid = lax.axis_index("s") * 2 + lax.axis_index("c")

        def chunk(j, carry):
            c = wid + j * NW
            s = pl.multiple_of(jnp.minimum(c * K, Bn - K), 8)

            @pl.when(c < C)
            def _():
                for t in range(n):
                    if not share_prev[t]:
                        pltpu.sync_copy(idr[t].at[pl.ds(s, K)], idx_v)
                    pltpu.async_copy(tabs[t].at[idx_v], rows_v, sem).wait()
                    pltpu.sync_copy(rows_v, outs[t].at[pl.ds(s, K)])

            return carry

        lax.fori_loop(0, per_w, chunk, 0)

    return pl.kernel(body, out_type=out_type, mesh=mesh,
                     scratch_types=scratch,
                     compiler_params=_SC_PARAMS)(*tables, *idxs)


def _sc_segsum(table, idxp, segp, starts, ends, npad, R, K=512):
    G = npad // NW
    S_n = G // R
    mesh = plsc.VectorSubcoreMesh(core_axis_name="c", subcore_axis_name="s")
    out_type = jax.ShapeDtypeStruct((npad * H,), _f32)
    scratch = [pltpu.VMEM((K,), jnp.int32), pltpu.VMEM((K,), jnp.int32),
               pltpu.VMEM((K,), jnp.int32),
               pltpu.VMEM((K, H), _f32),
               pltpu.VMEM(((R + 1) * H,), _f32),
               pltpu.VMEM((16,), jnp.int32), pltpu.VMEM((16,), jnp.int32),
               pltpu.SemaphoreType.DMA]

    def body(tab, idr, sgr, str_r, end_r, out, raw0, raw1, i0_v,
             rows_v, acc, sv, ev, sem):
        wid = lax.axis_index("s") * 2 + lax.axis_index("c")
        iota = lax.iota(jnp.int32, 16)
        zero16 = jnp.zeros((16,), _f32)

        def slab(p, carry):
            sl = wid * S_n + p
            base = sl * R
            c8 = pl.multiple_of((sl // 8) * 8, 8)
            pltpu.sync_copy(str_r.at[pl.ds(c8, 8)], sv.at[pl.ds(0, 8)])
            pltpu.sync_copy(end_r.at[pl.ds(c8, 8)], ev.at[pl.ds(0, 8)])
            lane = sl - c8
            e0 = jnp.sum(jnp.where(iota == lane, sv[...], 0))
            e1 = jnp.sum(jnp.where(iota == lane, ev[...], 0))

            def zrow(g, cc):
                b16 = g * 256
                for rr in range(16):
                    acc[pl.ds(b16 + rr * 16, 16)] = zero16
                return cc

            lax.fori_loop(0, R * H // 256, zrow, 0)
            nb = (e1 - e0 + K - 1) // K

            def blk(b, cc):
                s = pl.multiple_of(e0 + b * K, 8)
                pltpu.sync_copy(idr.at[pl.ds(s, K)], raw0)
                pltpu.sync_copy(sgr.at[pl.ds(s, K)], raw1)
                rem = e1 - s
                for k in range(K // 16):
                    sl16 = pl.ds(k * 16, 16)
                    local = raw1[sl16] - base
                    m = ((iota + (k * 16) < rem) & (local >= 0)
                         & (local < R))
                    i0_v[sl16] = jnp.where(m, raw0[sl16], 0)
                pltpu.async_copy(tab.at[i0_v], rows_v, sem).wait()
                for k in range(K // 16):
                    sl16 = pl.ds(k * 16, 16)
                    local = raw1[sl16] - base
                    m = ((iota + (k * 16) < rem) & (local >= 0)
                         & (local < R))
                    lv = jnp.where(m, local, R)
                    for j in range(16):
                        r = jnp.sum(jnp.where(iota == j, lv, 0))
                        pos = r * H + iota
                        for q in range(4):
                            x = rows_v[k * 16 + j, pl.ds(q * 16, 16)]
                            plsc.addupdate_scatter(
                                acc, [pos + (q * 16)], x)
                return cc

            lax.fori_loop(0, nb, blk, 0)
            pltpu.sync_copy(
                acc.at[pl.ds(0, R * H)],
                out.at[pl.ds(pl.multiple_of(base * H, 8), R * H)])
            return carry

        lax.fori_loop(0, S_n, slab, 0)

    out = pl.kernel(body, out_type=out_type, mesh=mesh,
                    scratch_types=scratch,
                    compiler_params=_SC_PARAMS)(table, idxp, segp, starts,
                                                ends)
    return out.reshape(npad, H)


def _pad_entries(idx, seg):
    pad0 = jnp.zeros((SEG_PAD,), jnp.int32)
    padm = jnp.full((SEG_PAD,), -1, jnp.int32)
    return (jnp.concatenate([idx, pad0]), jnp.concatenate([seg, padm]))


def _chunk_bounds(seg_sorted, nchunk, CH):
    cb = jnp.searchsorted(
        seg_sorted, jnp.arange(nchunk + 1, dtype=jnp.int32) * CH
    ).astype(jnp.int32)
    starts = (cb[:-1] // 8) * 8
    ends = (cb[1:] // 8) * 8 + 8
    lpad = (nchunk // 8) * 8 + 16
    z = jnp.zeros((lpad - nchunk,), jnp.int32)
    return jnp.concatenate([starts, z]), jnp.concatenate([ends, z])


def _mlpa_body(x_ref, w1_ref, b1_ref, p1_ref, w2_ref, b2_ref, t1_ref, st_ref):
    i = pl.program_id(0)
    h = x_ref[...] @ w1_ref[...] + b1_ref[...]
    h = jnp.where(h >= 0, h, p1_ref[0, 0] * h)
    t = h @ w2_ref[...] + b2_ref[...]
    t1_ref[...] = t

    @pl.when(i == 0)
    def _():
        st_ref[...] = jnp.zeros_like(st_ref)

    st_ref[...] += _stat_rows(t)


def _mlpa(x, w1, b1, p1, w2, b2):
    return pl.pallas_call(
        _mlpa_body,
        grid=(N // BNR,),
        in_specs=[_rows((BNR, IN_F)), _full((IN_F, H)), _full((1, H)),
                  _full((1, 1)), _full((H, H)), _full((1, H))],
        out_specs=[_rows((BNR, H)), _full((8, H))],
        out_shape=[jax.ShapeDtypeStruct((N, H), _f32),
                   jax.ShapeDtypeStruct((8, H), _f32)],
    )(x, w1, b1.reshape(1, H), p1.reshape(1, 1), w2, b2.reshape(1, H))


def _mlpb_body(t1_ref, sc_ref, sh_ref, p2_ref, w3_ref, b3_ref, t2_ref, st_ref):
    i = pl.program_id(0)
    u = t1_ref[...] * sc_ref[...] + sh_ref[...]
    u = jnp.where(u >= 0, u, p2_ref[0, 0] * u)
    t = u @ w3_ref[...] + b3_ref[...]
    t2_ref[...] = t

    @pl.when(i == 0)
    def _():
        st_ref[...] = jnp.zeros_like(st_ref)

    st_ref[...] += _stat_rows(t)


def _mlpb(t1, sc, sh, p2, w3, b3):
    return pl.pallas_call(
        _mlpb_body,
        grid=(N // BNR,),
        in_specs=[_rows((BNR, H)), _full((1, H)), _full((1, H)),
                  _full((1, 1)), _full((H, H)), _full((1, H))],
        out_specs=[_rows((BNR, H)), _full((8, H))],
        out_shape=[jax.ShapeDtypeStruct((N, H), _f32),
                   jax.ShapeDtypeStruct((8, H), _f32)],
    )(t1, sc.reshape(1, H), sh.reshape(1, H), p2.reshape(1, 1), w3,
      b3.reshape(1, H))


def _mlpc_body(t2_ref, sc_ref, sh_ref, wi_ref, wj_ref, h_ref, hi_ref, hj_ref):
    hh = t2_ref[...] * sc_ref[...] + sh_ref[...]
    h_ref[...] = hh
    hi_ref[...] = hh @ wi_ref[...]
    hj_ref[...] = hh @ wj_ref[...]


def _mlpc(t2, sc, sh, wi, wj):
    return pl.pallas_call(
        _mlpc_body,
        grid=(N // BNR,),
        in_specs=[_rows((BNR, H)), _full((1, H)), _full((1, H)),
                  _full((H, H)), _full((H, H))],
        out_specs=[_rows((BNR, H))] * 3,
        out_shape=[jax.ShapeDtypeStruct((N, H), _f32)] * 3,
    )(t2, sc.reshape(1, H), sh.reshape(1, H), wi, wj)


def _edge_body(hid_ref, hjs_ref, hs_ref, ef_ref, bb_ref, sp_ref, sw_ref,
               sb_ref, eew_ref, eeb_ref, ea_ref, ew_ref):
    a = hid_ref[...] + hjs_ref[...] + bb_ref[...]
    a = jnp.where(a >= 0, a, sp_ref[0, 0] * a)
    t = a @ sw_ref[...] + sb_ref[...]
    ef = ef_ref[...] @ eew_ref[...] + eeb_ref[...]
    alpha = (t * ef).sum(-1, keepdims=True) * (1.0 / 16.0)
    ew = jax.nn.sigmoid(alpha)
    ew_ref[...] = ew
    ea_ref[...] = hs_ref[...] * ew


def _edge_gate(HiD, HjS, Hs, edge_feats, bb, sp, sw, sb, eew, eeb):
    return pl.pallas_call(
        _edge_body,
        grid=(E // BE,),
        in_specs=[_rows((BE, H)), _rows((BE, H)), _rows((BE, H)),
                  _rows((BE, ED)),
                  _full((1, H)), _full((1, 1)), _full((H, H)), _full((1, H)),
                  _full((ED, H)), _full((1, H))],
        out_specs=[_rows((BE, H)), _rows((BE, 1))],
        out_shape=[jax.ShapeDtypeStruct((E, H), _f32),
                   jax.ShapeDtypeStruct((E, 1), _f32)],
    )(HiD, HjS, Hs, edge_feats, bb.reshape(1, H), sp.reshape(1, 1), sw,
      sb.reshape(1, H), eew, eeb.reshape(1, H))


def _upd_body(ea_ref, agg_ref, ew_ref, out_ref):
    out_ref[...] = ea_ref[...] + agg_ref[...] * ew_ref[...]


def _lg_update(ea, agg, ew):
    return pl.pallas_call(
        _upd_body,
        grid=(E // BE,),
        in_specs=[_rows((BE, H)), _rows((BE, H)), _rows((BE, 1))],
        out_specs=_rows((BE, H)),
        out_shape=jax.ShapeDtypeStruct((E, H), _f32),
    )(ea, agg, ew)


def _f1_body(h_ref, ag_ref, h2_ref, st_ref):
    i = pl.program_id(0)
    t = h_ref[...] + ag_ref[...]
    h2_ref[...] = t

    @pl.when(i == 0)
    def _():
        st_ref[...] = jnp.zeros_like(st_ref)

    st_ref[...] += _stat_rows(t)


def _f1(h, aggh):
    return pl.pallas_call(
        _f1_body,
        grid=(N // BNR,),
        in_specs=[_rows((BNR, H)), _rows((BNR, H))],
        out_specs=[_rows((BNR, H)), _full((8, H))],
        out_shape=[jax.ShapeDtypeStruct((N, H), _f32),
                   jax.ShapeDtypeStruct((8, H), _f32)],
    )(h, aggh)


def _lin_body(zin_ref, sc_ref, sh_ref, p_ref, w_ref, wb_ref, z_ref, st_ref):
    i = pl.program_id(0)
    u = zin_ref[...] * sc_ref[...] + sh_ref[...]
    u = jnp.where(u >= 0, u, p_ref[0, 0] * u)
    t = u @ w_ref[...] + wb_ref[...]
    z_ref[...] = t

    @pl.when(i == 0)
    def _():
        st_ref[...] = jnp.zeros_like(st_ref)

    st_ref[...] += _stat_rows(t)


def _lin(zin, sc, sh, p, w, wb, fin, fout):
    return pl.pallas_call(
        _lin_body,
        grid=(N // BNR,),
        in_specs=[_rows((BNR, fin)), _full((1, fin)), _full((1, fin)),
                  _full((1, 1)), _full((fin, fout)), _full((1, fout))],
        out_specs=[_rows((BNR, fout)), _full((8, fout))],
        out_shape=[jax.ShapeDtypeStruct((N, fout), _f32),
                   jax.ShapeDtypeStruct((8, fout), _f32)],
    )(zin, sc.reshape(1, fin), sh.reshape(1, fin), p.reshape(1, 1), w,
      wb.reshape(1, fout))


def _mix_body(za_ref, zb_ref, z_ref, st_ref):
    i = pl.program_id(0)
    t = (za_ref[...] + zb_ref[...]) * 0.5
    z_ref[...] = t

    @pl.when(i == 0)
    def _():
        st_ref[...] = jnp.zeros_like(st_ref)

    st_ref[...] += _stat_rows(t)


def _mix(za, zb):
    return pl.pallas_call(
        _mix_body,
        grid=(N // BNR,),
        in_specs=[_rows((BNR, S)), _rows((BNR, S))],
        out_specs=[_rows((BNR, S)), _full((8, S))],
        out_shape=[jax.ShapeDtypeStruct((N, S), _f32),
                   jax.ShapeDtypeStruct((8, S), _f32)],
    )(za, zb)


def _pairproj_body(xj_ref, xi_ref, wk_ref, wq_ref, wip_ref, wjp_ref,
                   kj_ref, qi_ref, pi_ref, pj_ref):
    xj = xj_ref[...]
    xi = xi_ref[...]
    kj_ref[...] = xj @ wk_ref[...]
    qi_ref[...] = xi @ wq_ref[...]
    pi_ref[...] = xi @ wip_ref[...]
    pj_ref[...] = xj @ wjp_ref[...]


def _pairproj(xj, xi, wk, wq, wip, wjp):
    nj = xj.shape[0]
    return pl.pallas_call(
        _pairproj_body,
        grid=(nj // BJ,),
        in_specs=[_rows((BJ, S)), _rows((BJ, S))] + [_full((S, H))] * 4,
        out_specs=[_rows((BJ, H))] * 4,
        out_shape=[jax.ShapeDtypeStruct((nj, H), _f32)] * 4,
    )(xj, xi, wk, wq, wip, wjp)


def _pairatt_body(kj_ref, qi_ref, pi_ref, pj_ref, cb_ref, ca_ref, pv_ref):
    t = jnp.tanh(kj_ref[...] + qi_ref[...] + cb_ref[...])
    att = (t * ca_ref[...]).sum(-1, keepdims=True)
    pv_ref[...] = att * (pi_ref[...] * pj_ref[...])


def _pairatt(KjP, QiP, PiP, PjP, cb, ca):
    return pl.pallas_call(
        _pairatt_body,
        grid=(PE // BP,),
        in_specs=[_rows((BP, H))] * 4 + [_full((1, H)), _full((1, H))],
        out_specs=_rows((BP, H)),
        out_shape=jax.ShapeDtypeStruct((PE, H), _f32),
    )(KjP, QiP, PiP, PjP, cb.reshape(1, H), ca.reshape(1, H))


def _score_body(pg_ref, rg_ref, s_ref):
    s_ref[...] = (pg_ref[...] * rg_ref[...]).sum(-1, keepdims=True)


def _score(pair_g, r_g):
    n = pair_g.shape[0]
    return pl.pallas_call(
        _score_body,
        in_specs=[_full((n, H)), _full((n, H))],
        out_specs=_full((n, 1)),
        out_shape=jax.ShapeDtypeStruct((n, 1), _f32),
    )(pair_g, r_g)


def kernel(x, edge_feats, params, edge_index, line_graph_edge_index,
           pair_edge_index, edge_index_batch, rels, drug_pair_indices,
           node_j_for_pairs, node_i_for_pairs):
    p = params
    src, dst = edge_index[0], edge_index[1]

    t1, st1 = _mlpa(x, p['mlp_w1'], p['mlp_b1'], p['mlp_p1'], p['mlp_w2'],
                    p['mlp_b2'])
    sc1, sh1 = _bn_coeff(st1, N, p['mlp_bn1_g'], p['mlp_bn1_b'])
    t2, st2 = _mlpb(t1, sc1, sh1, p['mlp_p2'], p['mlp_w3'], p['mlp_b3'])
    sc2, sh2 = _bn_coeff(st2, N, p['mlp_bn2_g'], p['mlp_bn2_b'])
    h, hi, hj = _mlpc(t2, sc2, sh2, p['w_i'], p['w_j'])

    HiD, HjS, Hs = _sc_multigather([hi, hj, h], [dst, src, src],
                                   [False, False, True], 1000)
    ea, ew = _edge_gate(HiD, HjS, Hs, edge_feats, p['blk_bias'], p['sml_p'],
                        p['sml_w'], p['sml_b'], p['ee_w'], p['ee_b'])

    lg1s, lg0s = lax.sort((line_graph_edge_index[1], line_graph_edge_index[0]),
                          num_keys=1)
    lg0p, lg1p = _pad_entries(lg0s, lg1s)
    lst, len_ = _chunk_bounds(lg1s, 800, 1000)
    out = ea
    for _ in range(N_ITER):
        agg = _sc_segsum(out, lg0p, lg1p, lst, len_, E, 1000)
        out = _lg_update(ea, agg, ew)

    dsts, eids = lax.sort((dst, jnp.arange(E, dtype=dst.dtype)), num_keys=1)
    eidp, dstp = _pad_entries(eids, dsts)
    nst, nen = _chunk_bounds(dsts, 64, 1000)
    aggh = _sc_segsum(out, eidp, dstp, nst, nen, 64000, 1000)[:N]

    h2, sth = _f1(h, aggh)
    scl1, shl1 = _bn_coeff(sth, N, p['l1_g'], p['l1_b'])
    z, stz = _lin(h2, scl1, shl1, jnp.float32(1.0), p['l1_w'], p['l1_wb'],
                  H, S)
    scz, shz = _bn_coeff(stz, N, p['l2_g'], p['l2_b'])
    z2, st22 = _lin(z, scz, shz, p['l2_p'], p['l2_w'], p['l2_wb'], S, S)
    sc3, sh3 = _bn_coeff(st22, N, p['l3_g'], p['l3_b'])
    z3, _ = _lin(z2, sc3, sh3, p['l3_p'], p['l3_w'], p['l3_wb'], S, S)
    zB, stB = _mix(z3, z)
    sc4, sh4 = _bn_coeff(stB, N, p['l4_g'], p['l4_b'])
    z4, _ = _lin(zB, sc4, sh4, p['l4_p'], p['l4_w'], p['l4_wb'], S, S)
    zC, _ = _mix(z4, zB)

    xj, xi = _sc_multigather([zC, zC], [node_j_for_pairs, node_i_for_pairs],
                             [False, False], 512)
    Kj, Qi, Pi, Pj = _pairproj(xj, xi, p['ca_wk'], p['ca_wq'], p['i_pro'],
                               p['j_pro'])
    pe0, pe1 = pair_edge_index[0], pair_edge_index[1]
    KjP, QiP, PiP, PjP = _sc_multigather(
        [Kj, Qi, Pi, Pj], [pe0, pe1, pe1, pe0],
        [False, False, True, False], 1000)
    pv = _pairatt(KjP, QiP, PiP, PjP, p['ca_bias'], p['ca_a'])
    arp, eibp = _pad_entries(jnp.arange(PE, dtype=jnp.int32),
                             edge_index_batch)
    pst, pen = _chunk_bounds(edge_index_batch, 32, 64)
    seg = _sc_segsum(pv, arp, eibp, pst, pen, NPAIRS, 64)
    rels_full = jnp.concatenate([rels, rels])
    pair_g, r_g = _sc_multigather([seg, p['rel_embs']],
                                  [drug_pair_indices, rels_full],
                                  [False, False], 64)
    scores = _score(pair_g, r_g)
    return scores[:B], scores[B:].reshape(B, 1, 1)


# speedup vs baseline: 2.2263x; 1.1786x over previous
import functools

import jax
import jax.numpy as jnp
from jax import lax
from jax.experimental import pallas as pl
from jax.experimental.pallas import tpu as pltpu
from jax.experimental.pallas import tpu_sc as plsc

N, E, LE, PE = 50000, 800000, 1200000, 400000
IN_F, H, S, ED = 128, 64, 128, 16
NPAIRS, B = 2048, 1024
N_ITER = 3

BNR = 2000
BE = 8000
BP = 8000
BJ = 4000

_f32 = jnp.float32


def _full(shape):
    return pl.BlockSpec(shape, lambda *a: tuple(0 for _ in shape))


def _rows(bshape):
    return pl.BlockSpec(bshape, lambda i: (i,) + tuple(0 for _ in bshape[1:]))


def _stat_rows(t):
    return jnp.concatenate(
        [t.sum(0, keepdims=True), (t * t).sum(0, keepdims=True),
         jnp.zeros((6, t.shape[1]), _f32)], axis=0)


def _bn_coeff(st, n, g, b, eps=1e-5):
    m = st[0] / n
    v = st[1] / n - m * m
    inv = g / jnp.sqrt(v + eps)
    return inv, b - m * inv


NW = 32
SEG_PAD = 2560
_SC_PARAMS = pltpu.CompilerParams(use_tc_tiling_on_sc=False,
                                  needs_layout_passes=False)


def _sc_multigather(tables, idxs, share_prev, K):
    n = len(tables)
    Bn = idxs[0].shape[0]
    D = tables[0].shape[1]
    C = -(-Bn // K)
    per_w = -(-C // NW)
    mesh = plsc.VectorSubcoreMesh(core_axis_name="c", subcore_axis_name="s")
    out_type = [jax.ShapeDtypeStruct((Bn, D), _f32) for _ in range(n)]
    scratch = [pltpu.VMEM((K,), jnp.int32), pltpu.VMEM((K, D), _f32),
               pltpu.SemaphoreType.DMA]

    def body(*refs):
        tabs = refs[:n]
        idr = refs[n:2 * n]
        outs = refs[2 * n:3 * n]
        idx_v, rows_v, sem = refs[3 * n:]
        wid = lax.axis_index("s") * 2 + lax.axis_index("c")

        def chunk(j, carry):
            c = wid + j * NW
            s = pl.multiple_of(jnp.minimum(c * K, Bn - K), 8)

            @pl.when(c < C)
            def _():
                for t in range(n):
                    if not share_prev[t]:
                        pltpu.sync_copy(idr[t].at[pl.ds(s, K)], idx_v)
                    pltpu.async_copy(tabs[t].at[idx_v], rows_v, sem).wait()
                    pltpu.sync_copy(rows_v, outs[t].at[pl.ds(s, K)])

            return carry

        lax.fori_loop(0, per_w, chunk, 0)

    return pl.kernel(body, out_type=out_type, mesh=mesh,
                     scratch_types=scratch,
                     compiler_params=_SC_PARAMS)(*tables, *idxs)


def _sc_segsum(table, idxp, segp, starts, ends, npad, R, K=256):
    G = npad // NW
    S_n = G // R
    mesh = plsc.VectorSubcoreMesh(core_axis_name="c", subcore_axis_name="s")
    out_type = jax.ShapeDtypeStruct((npad * H,), _f32)
    scratch = [pltpu.VMEM((K,), jnp.int32),
               pltpu.VMEM((K,), jnp.int32), pltpu.VMEM((K,), jnp.int32),
               pltpu.VMEM((K,), jnp.int32), pltpu.VMEM((K,), jnp.int32),
               pltpu.VMEM((K, H), _f32), pltpu.VMEM((K, H), _f32),
               pltpu.VMEM(((R + 1) * H,), _f32),
               pltpu.VMEM((16,), jnp.int32), pltpu.VMEM((16,), jnp.int32),
               pltpu.SemaphoreType.DMA, pltpu.SemaphoreType.DMA]

    def body(tab, idr, sgr, str_r, end_r, out, raw0, raw1a, raw1b, i0a, i0b,
             rows_a, rows_b, acc, sv, ev, sem_a, sem_b):
        raw1s, i0s, rowss, sems = ((raw1a, raw1b), (i0a, i0b),
                                   (rows_a, rows_b), (sem_a, sem_b))
        wid = lax.axis_index("s") * 2 + lax.axis_index("c")
        iota = lax.iota(jnp.int32, 16)
        zero16 = jnp.zeros((16,), _f32)

        def slab(p, carry):
            sl = wid * S_n + p
            base = sl * R
            c8 = pl.multiple_of((sl // 8) * 8, 8)
            pltpu.sync_copy(str_r.at[pl.ds(c8, 8)], sv.at[pl.ds(0, 8)])
            pltpu.sync_copy(end_r.at[pl.ds(c8, 8)], ev.at[pl.ds(0, 8)])
            lane = sl - c8
            e0 = jnp.sum(jnp.where(iota == lane, sv[...], 0))
            e1 = jnp.sum(jnp.where(iota == lane, ev[...], 0))

            def zrow(g, cc):
                b16 = g * 256
                for rr in range(16):
                    acc[pl.ds(b16 + rr * 16, 16)] = zero16
                return cc

            lax.fori_loop(0, R * H // 256, zrow, 0)
            nb = (e1 - e0 + K - 1) // K

            def stage(b, buf):
                s = pl.multiple_of(e0 + b * K, 8)
                pltpu.sync_copy(idr.at[pl.ds(s, K)], raw0)
                pltpu.sync_copy(sgr.at[pl.ds(s, K)], raw1s[buf])
                rem = e1 - s
                for k in range(K // 16):
                    sl16 = pl.ds(k * 16, 16)
                    local = raw1s[buf][sl16] - base
                    m = ((iota + (k * 16) < rem) & (local >= 0)
                         & (local < R))
                    i0s[buf][sl16] = jnp.where(m, raw0[sl16], 0)
                pltpu.async_copy(tab.at[i0s[buf]], rowss[buf], sems[buf])

            def addblk(b, buf):
                s = e0 + b * K
                rem = e1 - s
                pltpu.make_async_copy(tab.at[i0s[buf]], rowss[buf],
                                      sems[buf]).wait()
                for k in range(K // 16):
                    sl16 = pl.ds(k * 16, 16)
                    local = raw1s[buf][sl16] - base
                    m = ((iota + (k * 16) < rem) & (local >= 0)
                         & (local < R))
                    lv = jnp.where(m, local, R)
                    for j in range(16):
                        r = jnp.sum(jnp.where(iota == j, lv, 0))
                        pos = r * H + iota
                        for q in range(4):
                            x = rowss[buf][k * 16 + j, pl.ds(q * 16, 16)]
                            plsc.addupdate_scatter(
                                acc, [pos + (q * 16)], x)

            @pl.when(nb > 0)
            def _():
                stage(0, 0)

            def pair_blk(g, cc):
                for q in (0, 1):
                    b = 2 * g + q

                    @pl.when(b < nb)
                    def _():
                        @pl.when(b + 1 < nb)
                        def _():
                            stage(b + 1, 1 - q)

                        addblk(b, q)

                return cc

            lax.fori_loop(0, (nb + 1) // 2, pair_blk, 0)
            pltpu.sync_copy(
                acc.at[pl.ds(0, R * H)],
                out.at[pl.ds(pl.multiple_of(base * H, 8), R * H)])
            return carry

        lax.fori_loop(0, S_n, slab, 0)

    out = pl.kernel(body, out_type=out_type, mesh=mesh,
                    scratch_types=scratch,
                    compiler_params=_SC_PARAMS)(table, idxp, segp, starts,
                                                ends)
    return out.reshape(npad, H)


def _pad_entries(idx, seg):
    pad0 = jnp.zeros((SEG_PAD,), jnp.int32)
    padm = jnp.full((SEG_PAD,), -1, jnp.int32)
    return (jnp.concatenate([idx, pad0]), jnp.concatenate([seg, padm]))


def _chunk_bounds(seg_sorted, nchunk, CH):
    cb = jnp.searchsorted(
        seg_sorted, jnp.arange(nchunk + 1, dtype=jnp.int32) * CH
    ).astype(jnp.int32)
    starts = (cb[:-1] // 8) * 8
    ends = (cb[1:] // 8) * 8 + 8
    lpad = (nchunk // 8) * 8 + 16
    z = jnp.zeros((lpad - nchunk,), jnp.int32)
    return jnp.concatenate([starts, z]), jnp.concatenate([ends, z])


def _mlpa_body(x_ref, w1_ref, b1_ref, p1_ref, w2_ref, b2_ref, t1_ref, st_ref):
    i = pl.program_id(0)
    h = x_ref[...] @ w1_ref[...] + b1_ref[...]
    h = jnp.where(h >= 0, h, p1_ref[0, 0] * h)
    t = h @ w2_ref[...] + b2_ref[...]
    t1_ref[...] = t

    @pl.when(i == 0)
    def _():
        st_ref[...] = jnp.zeros_like(st_ref)

    st_ref[...] += _stat_rows(t)


def _mlpa(x, w1, b1, p1, w2, b2):
    return pl.pallas_call(
        _mlpa_body,
        grid=(N // BNR,),
        in_specs=[_rows((BNR, IN_F)), _full((IN_F, H)), _full((1, H)),
                  _full((1, 1)), _full((H, H)), _full((1, H))],
        out_specs=[_rows((BNR, H)), _full((8, H))],
        out_shape=[jax.ShapeDtypeStruct((N, H), _f32),
                   jax.ShapeDtypeStruct((8, H), _f32)],
    )(x, w1, b1.reshape(1, H), p1.reshape(1, 1), w2, b2.reshape(1, H))


def _mlpb_body(t1_ref, sc_ref, sh_ref, p2_ref, w3_ref, b3_ref, t2_ref, st_ref):
    i = pl.program_id(0)
    u = t1_ref[...] * sc_ref[...] + sh_ref[...]
    u = jnp.where(u >= 0, u, p2_ref[0, 0] * u)
    t = u @ w3_ref[...] + b3_ref[...]
    t2_ref[...] = t

    @pl.when(i == 0)
    def _():
        st_ref[...] = jnp.zeros_like(st_ref)

    st_ref[...] += _stat_rows(t)


def _mlpb(t1, sc, sh, p2, w3, b3):
    return pl.pallas_call(
        _mlpb_body,
        grid=(N // BNR,),
        in_specs=[_rows((BNR, H)), _full((1, H)), _full((1, H)),
                  _full((1, 1)), _full((H, H)), _full((1, H))],
        out_specs=[_rows((BNR, H)), _full((8, H))],
        out_shape=[jax.ShapeDtypeStruct((N, H), _f32),
                   jax.ShapeDtypeStruct((8, H), _f32)],
    )(t1, sc.reshape(1, H), sh.reshape(1, H), p2.reshape(1, 1), w3,
      b3.reshape(1, H))


def _mlpc_body(t2_ref, sc_ref, sh_ref, wi_ref, wj_ref, h_ref, hi_ref, hj_ref):
    hh = t2_ref[...] * sc_ref[...] + sh_ref[...]
    h_ref[...] = hh
    hi_ref[...] = hh @ wi_ref[...]
    hj_ref[...] = hh @ wj_ref[...]


def _mlpc(t2, sc, sh, wi, wj):
    return pl.pallas_call(
        _mlpc_body,
        grid=(N // BNR,),
        in_specs=[_rows((BNR, H)), _full((1, H)), _full((1, H)),
                  _full((H, H)), _full((H, H))],
        out_specs=[_rows((BNR, H))] * 3,
        out_shape=[jax.ShapeDtypeStruct((N, H), _f32)] * 3,
    )(t2, sc.reshape(1, H), sh.reshape(1, H), wi, wj)


def _edge_body(hid_ref, hjs_ref, hs_ref, ef_ref, bb_ref, sp_ref, sw_ref,
               sb_ref, eew_ref, eeb_ref, ea_ref, ew_ref):
    a = hid_ref[...] + hjs_ref[...] + bb_ref[...]
    a = jnp.where(a >= 0, a, sp_ref[0, 0] * a)
    t = a @ sw_ref[...] + sb_ref[...]
    ef = ef_ref[...] @ eew_ref[...] + eeb_ref[...]
    alpha = (t * ef).sum(-1, keepdims=True) * (1.0 / 16.0)
    ew = jax.nn.sigmoid(alpha)
    ew_ref[...] = ew
    ea_ref[...] = hs_ref[...] * ew


def _edge_gate(HiD, HjS, Hs, edge_feats, bb, sp, sw, sb, eew, eeb):
    return pl.pallas_call(
        _edge_body,
        grid=(E // BE,),
        in_specs=[_rows((BE, H)), _rows((BE, H)), _rows((BE, H)),
                  _rows((BE, ED)),
                  _full((1, H)), _full((1, 1)), _full((H, H)), _full((1, H)),
                  _full((ED, H)), _full((1, H))],
        out_specs=[_rows((BE, H)), _rows((BE, 1))],
        out_shape=[jax.ShapeDtypeStruct((E, H), _f32),
                   jax.ShapeDtypeStruct((E, 1), _f32)],
    )(HiD, HjS, Hs, edge_feats, bb.reshape(1, H), sp.reshape(1, 1), sw,
      sb.reshape(1, H), eew, eeb.reshape(1, H))


def _upd_body(ea_ref, agg_ref, ew_ref, out_ref):
    out_ref[...] = ea_ref[...] + agg_ref[...] * ew_ref[...]


def _lg_update(ea, agg, ew):
    return pl.pallas_call(
        _upd_body,
        grid=(E // BE,),
        in_specs=[_rows((BE, H)), _rows((BE, H)), _rows((BE, 1))],
        out_specs=_rows((BE, H)),
        out_shape=jax.ShapeDtypeStruct((E, H), _f32),
    )(ea, agg, ew)


def _f1_body(h_ref, ag_ref, h2_ref, st_ref):
    i = pl.program_id(0)
    t = h_ref[...] + ag_ref[...]
    h2_ref[...] = t

    @pl.when(i == 0)
    def _():
        st_ref[...] = jnp.zeros_like(st_ref)

    st_ref[...] += _stat_rows(t)


def _f1(h, aggh):
    return pl.pallas_call(
        _f1_body,
        grid=(N // BNR,),
        in_specs=[_rows((BNR, H)), _rows((BNR, H))],
        out_specs=[_rows((BNR, H)), _full((8, H))],
        out_shape=[jax.ShapeDtypeStruct((N, H), _f32),
                   jax.ShapeDtypeStruct((8, H), _f32)],
    )(h, aggh)


def _lin_body(zin_ref, sc_ref, sh_ref, p_ref, w_ref, wb_ref, z_ref, st_ref):
    i = pl.program_id(0)
    u = zin_ref[...] * sc_ref[...] + sh_ref[...]
    u = jnp.where(u >= 0, u, p_ref[0, 0] * u)
    t = u @ w_ref[...] + wb_ref[...]
    z_ref[...] = t

    @pl.when(i == 0)
    def _():
        st_ref[...] = jnp.zeros_like(st_ref)

    st_ref[...] += _stat_rows(t)


def _lin(zin, sc, sh, p, w, wb, fin, fout):
    return pl.pallas_call(
        _lin_body,
        grid=(N // BNR,),
        in_specs=[_rows((BNR, fin)), _full((1, fin)), _full((1, fin)),
                  _full((1, 1)), _full((fin, fout)), _full((1, fout))],
        out_specs=[_rows((BNR, fout)), _full((8, fout))],
        out_shape=[jax.ShapeDtypeStruct((N, fout), _f32),
                   jax.ShapeDtypeStruct((8, fout), _f32)],
    )(zin, sc.reshape(1, fin), sh.reshape(1, fin), p.reshape(1, 1), w,
      wb.reshape(1, fout))


def _mix_body(za_ref, zb_ref, z_ref, st_ref):
    i = pl.program_id(0)
    t = (za_ref[...] + zb_ref[...]) * 0.5
    z_ref[...] = t

    @pl.when(i == 0)
    def _():
        st_ref[...] = jnp.zeros_like(st_ref)

    st_ref[...] += _stat_rows(t)


def _mix(za, zb):
    return pl.pallas_call(
        _mix_body,
        grid=(N // BNR,),
        in_specs=[_rows((BNR, S)), _rows((BNR, S))],
        out_specs=[_rows((BNR, S)), _full((8, S))],
        out_shape=[jax.ShapeDtypeStruct((N, S), _f32),
                   jax.ShapeDtypeStruct((8, S), _f32)],
    )(za, zb)


def _pairproj_body(xj_ref, xi_ref, wk_ref, wq_ref, wip_ref, wjp_ref,
                   kj_ref, qi_ref, pi_ref, pj_ref):
    xj = xj_ref[...]
    xi = xi_ref[...]
    kj_ref[...] = xj @ wk_ref[...]
    qi_ref[...] = xi @ wq_ref[...]
    pi_ref[...] = xi @ wip_ref[...]
    pj_ref[...] = xj @ wjp_ref[...]


def _pairproj(xj, xi, wk, wq, wip, wjp):
    nj = xj.shape[0]
    return pl.pallas_call(
        _pairproj_body,
        grid=(nj // BJ,),
        in_specs=[_rows((BJ, S)), _rows((BJ, S))] + [_full((S, H))] * 4,
        out_specs=[_rows((BJ, H))] * 4,
        out_shape=[jax.ShapeDtypeStruct((nj, H), _f32)] * 4,
    )(xj, xi, wk, wq, wip, wjp)


def _pairatt_body(kj_ref, qi_ref, pi_ref, pj_ref, cb_ref, ca_ref, pv_ref):
    t = jnp.tanh(kj_ref[...] + qi_ref[...] + cb_ref[...])
    att = (t * ca_ref[...]).sum(-1, keepdims=True)
    pv_ref[...] = att * (pi_ref[...] * pj_ref[...])


def _pairatt(KjP, QiP, PiP, PjP, cb, ca):
    return pl.pallas_call(
        _pairatt_body,
        grid=(PE // BP,),
        in_specs=[_rows((BP, H))] * 4 + [_full((1, H)), _full((1, H))],
        out_specs=_rows((BP, H)),
        out_shape=jax.ShapeDtypeStruct((PE, H), _f32),
    )(KjP, QiP, PiP, PjP, cb.reshape(1, H), ca.reshape(1, H))


def _score_body(pg_ref, rg_ref, s_ref):
    s_ref[...] = (pg_ref[...] * rg_ref[...]).sum(-1, keepdims=True)


def _score(pair_g, r_g):
    n = pair_g.shape[0]
    return pl.pallas_call(
        _score_body,
        in_specs=[_full((n, H)), _full((n, H))],
        out_specs=_full((n, 1)),
        out_shape=jax.ShapeDtypeStruct((n, 1), _f32),
    )(pair_g, r_g)


def kernel(x, edge_feats, params, edge_index, line_graph_edge_index,
           pair_edge_index, edge_index_batch, rels, drug_pair_indices,
           node_j_for_pairs, node_i_for_pairs):
    p = params
    src, dst = edge_index[0], edge_index[1]

    t1, st1 = _mlpa(x, p['mlp_w1'], p['mlp_b1'], p['mlp_p1'], p['mlp_w2'],
                    p['mlp_b2'])
    sc1, sh1 = _bn_coeff(st1, N, p['mlp_bn1_g'], p['mlp_bn1_b'])
    t2, st2 = _mlpb(t1, sc1, sh1, p['mlp_p2'], p['mlp_w3'], p['mlp_b3'])
    sc2, sh2 = _bn_coeff(st2, N, p['mlp_bn2_g'], p['mlp_bn2_b'])
    h, hi, hj = _mlpc(t2, sc2, sh2, p['w_i'], p['w_j'])

    HiD, HjS, Hs = _sc_multigather([hi, hj, h], [dst, src, src],
                                   [False, False, True], 1000)
    ea, ew = _edge_gate(HiD, HjS, Hs, edge_feats, p['blk_bias'], p['sml_p'],
                        p['sml_w'], p['sml_b'], p['ee_w'], p['ee_b'])

    lg1s, lg0s = lax.sort((line_graph_edge_index[1], line_graph_edge_index[0]),
                          num_keys=1)
    lg0p, lg1p = _pad_entries(lg0s, lg1s)
    lst, len_ = _chunk_bounds(lg1s, 800, 1000)
    out = ea
    for _ in range(N_ITER):
        agg = _sc_segsum(out, lg0p, lg1p, lst, len_, E, 1000)
        out = _lg_update(ea, agg, ew)

    dsts, eids = lax.sort((dst, jnp.arange(E, dtype=dst.dtype)), num_keys=1)
    eidp, dstp = _pad_entries(eids, dsts)
    nst, nen = _chunk_bounds(dsts, 64, 1000)
    aggh = _sc_segsum(out, eidp, dstp, nst, nen, 64000, 1000)[:N]

    h2, sth = _f1(h, aggh)
    scl1, shl1 = _bn_coeff(sth, N, p['l1_g'], p['l1_b'])
    z, stz = _lin(h2, scl1, shl1, jnp.float32(1.0), p['l1_w'], p['l1_wb'],
                  H, S)
    scz, shz = _bn_coeff(stz, N, p['l2_g'], p['l2_b'])
    z2, st22 = _lin(z, scz, shz, p['l2_p'], p['l2_w'], p['l2_wb'], S, S)
    sc3, sh3 = _bn_coeff(st22, N, p['l3_g'], p['l3_b'])
    z3, _ = _lin(z2, sc3, sh3, p['l3_p'], p['l3_w'], p['l3_wb'], S, S)
    zB, stB = _mix(z3, z)
    sc4, sh4 = _bn_coeff(stB, N, p['l4_g'], p['l4_b'])
    z4, _ = _lin(zB, sc4, sh4, p['l4_p'], p['l4_w'], p['l4_wb'], S, S)
    zC, _ = _mix(z4, zB)

    xj, xi = _sc_multigather([zC, zC], [node_j_for_pairs, node_i_for_pairs],
                             [False, False], 512)
    Kj, Qi, Pi, Pj = _pairproj(xj, xi, p['ca_wk'], p['ca_wq'], p['i_pro'],
                               p['j_pro'])
    pe0, pe1 = pair_edge_index[0], pair_edge_index[1]
    KjP, QiP, PiP, PjP = _sc_multigather(
        [Kj, Qi, Pi, Pj], [pe0, pe1, pe1, pe0],
        [False, False, True, False], 1000)
    pv = _pairatt(KjP, QiP, PiP, PjP, p['ca_bias'], p['ca_a'])
    arp, eibp = _pad_entries(jnp.arange(PE, dtype=jnp.int32),
                             edge_index_batch)
    pst, pen = _chunk_bounds(edge_index_batch, 32, 64)
    seg = _sc_segsum(pv, arp, eibp, pst, pen, NPAIRS, 64)
    rels_full = jnp.concatenate([rels, rels])
    pair_g, r_g = _sc_multigather([seg, p['rel_embs']],
                                  [drug_pair_indices, rels_full],
                                  [False, False], 64)
    scores = _score(pair_g, r_g)
    return scores[:B], scores[B:].reshape(B, 1, 1)


# speedup vs baseline: 2.2278x; 1.0007x over previous
import functools

import jax
import jax.numpy as jnp
from jax import lax
from jax.experimental import pallas as pl
from jax.experimental.pallas import tpu as pltpu
from jax.experimental.pallas import tpu_sc as plsc

N, E, LE, PE = 50000, 800000, 1200000, 400000
IN_F, H, S, ED = 128, 64, 128, 16
NPAIRS, B = 2048, 1024
N_ITER = 3

BNR = 2000
BE = 8000
BP = 8000
BJ = 4000

_f32 = jnp.float32


def _full(shape):
    return pl.BlockSpec(shape, lambda *a: tuple(0 for _ in shape))


def _rows(bshape):
    return pl.BlockSpec(bshape, lambda i: (i,) + tuple(0 for _ in bshape[1:]))


def _stat_rows(t):
    return jnp.concatenate(
        [t.sum(0, keepdims=True), (t * t).sum(0, keepdims=True),
         jnp.zeros((6, t.shape[1]), _f32)], axis=0)


def _bn_coeff(st, n, g, b, eps=1e-5):
    m = st[0] / n
    v = st[1] / n - m * m
    inv = g / jnp.sqrt(v + eps)
    return inv, b - m * inv


NW = 32
SEG_PAD = 2560
_SC_PARAMS = pltpu.CompilerParams(use_tc_tiling_on_sc=False,
                                  needs_layout_passes=False)


def _sc_multigather(tables, idxs, idx_of, K):
    n = len(tables)
    ni = len(idxs)
    Bn = idxs[0].shape[0]
    D = tables[0].shape[1]
    C = -(-Bn // K)
    per_w = -(-C // NW)
    mesh = plsc.VectorSubcoreMesh(core_axis_name="c", subcore_axis_name="s")
    out_type = [jax.ShapeDtypeStruct((Bn, D), _f32) for _ in range(n)]
    scratch = ([pltpu.VMEM((K,), jnp.int32)] * ni
               + [pltpu.VMEM((K, D), _f32)] * n
               + [pltpu.SemaphoreType.DMA] * n)

    def body(*refs):
        tabs = refs[:n]
        idr = refs[n:n + ni]
        outs = refs[n + ni:2 * n + ni]
        idx_v = refs[2 * n + ni:2 * n + 2 * ni]
        rows_v = refs[2 * n + 2 * ni:3 * n + 2 * ni]
        sems = refs[3 * n + 2 * ni:]
        wid = lax.axis_index("s") * 2 + lax.axis_index("c")

        def chunk(j, carry):
            c = wid + j * NW
            s = pl.multiple_of(jnp.minimum(c * K, Bn - K), 8)

            @pl.when(c < C)
            def _():
                for i in range(ni):
                    pltpu.sync_copy(idr[i].at[pl.ds(s, K)], idx_v[i])
                for t in range(n):
                    pltpu.async_copy(tabs[t].at[idx_v[idx_of[t]]],
                                     rows_v[t], sems[t])
                for t in range(n):
                    pltpu.make_async_copy(tabs[t].at[idx_v[idx_of[t]]],
                                          rows_v[t], sems[t]).wait()
                    pltpu.sync_copy(rows_v[t], outs[t].at[pl.ds(s, K)])

            return carry

        lax.fori_loop(0, per_w, chunk, 0)

    return pl.kernel(body, out_type=out_type, mesh=mesh,
                     scratch_types=scratch,
                     compiler_params=_SC_PARAMS)(*tables, *idxs)


def _sc_segsum(table, idxp, segp, starts, ends, npad, R, K=256):
    G = npad // NW
    S_n = G // R
    mesh = plsc.VectorSubcoreMesh(core_axis_name="c", subcore_axis_name="s")
    out_type = jax.ShapeDtypeStruct((npad * H,), _f32)
    scratch = [pltpu.VMEM((K,), jnp.int32),
               pltpu.VMEM((K,), jnp.int32), pltpu.VMEM((K,), jnp.int32),
               pltpu.VMEM((K,), jnp.int32), pltpu.VMEM((K,), jnp.int32),
               pltpu.VMEM((K, H), _f32), pltpu.VMEM((K, H), _f32),
               pltpu.VMEM(((R + 1) * H,), _f32),
               pltpu.VMEM((16,), jnp.int32), pltpu.VMEM((16,), jnp.int32),
               pltpu.SemaphoreType.DMA, pltpu.SemaphoreType.DMA]

    def body(tab, idr, sgr, str_r, end_r, out, raw0, raw1a, raw1b, i0a, i0b,
             rows_a, rows_b, acc, sv, ev, sem_a, sem_b):
        raw1s, i0s, rowss, sems = ((raw1a, raw1b), (i0a, i0b),
                                   (rows_a, rows_b), (sem_a, sem_b))
        wid = lax.axis_index("s") * 2 + lax.axis_index("c")
        iota = lax.iota(jnp.int32, 16)
        zero16 = jnp.zeros((16,), _f32)

        def slab(p, carry):
            sl = wid * S_n + p
            base = sl * R
            c8 = pl.multiple_of((sl // 8) * 8, 8)
            pltpu.sync_copy(str_r.at[pl.ds(c8, 8)], sv.at[pl.ds(0, 8)])
            pltpu.sync_copy(end_r.at[pl.ds(c8, 8)], ev.at[pl.ds(0, 8)])
            lane = sl - c8
            e0 = jnp.sum(jnp.where(iota == lane, sv[...], 0))
            e1 = jnp.sum(jnp.where(iota == lane, ev[...], 0))

            def zrow(g, cc):
                b16 = g * 256
                for rr in range(16):
                    acc[pl.ds(b16 + rr * 16, 16)] = zero16
                return cc

            lax.fori_loop(0, R * H // 256, zrow, 0)
            nb = (e1 - e0 + K - 1) // K

            def stage(b, buf):
                s = pl.multiple_of(e0 + b * K, 8)
                pltpu.sync_copy(idr.at[pl.ds(s, K)], raw0)
                pltpu.sync_copy(sgr.at[pl.ds(s, K)], raw1s[buf])
                rem = e1 - s
                for k in range(K // 16):
                    sl16 = pl.ds(k * 16, 16)
                    local = raw1s[buf][sl16] - base
                    m = ((iota + (k * 16) < rem) & (local >= 0)
                         & (local < R))
                    i0s[buf][sl16] = jnp.where(m, raw0[sl16], 0)
                pltpu.async_copy(tab.at[i0s[buf]], rowss[buf], sems[buf])

            def addblk(b, buf):
                s = e0 + b * K
                rem = e1 - s
                pltpu.make_async_copy(tab.at[i0s[buf]], rowss[buf],
                                      sems[buf]).wait()
                for k in range(K // 16):
                    sl16 = pl.ds(k * 16, 16)
                    local = raw1s[buf][sl16] - base
                    m = ((iota + (k * 16) < rem) & (local >= 0)
                         & (local < R))
                    lv = jnp.where(m, local, R)
                    for j in range(16):
                        r = jnp.sum(jnp.where(iota == j, lv, 0))
                        pos = r * H + iota
                        for q in range(4):
                            x = rowss[buf][k * 16 + j, pl.ds(q * 16, 16)]
                            plsc.addupdate_scatter(
                                acc, [pos + (q * 16)], x)

            @pl.when(nb > 0)
            def _():
                stage(0, 0)

            def pair_blk(g, cc):
                for q in (0, 1):
                    b = 2 * g + q

                    @pl.when(b < nb)
                    def _():
                        @pl.when(b + 1 < nb)
                        def _():
                            stage(b + 1, 1 - q)

                        addblk(b, q)

                return cc

            lax.fori_loop(0, (nb + 1) // 2, pair_blk, 0)
            pltpu.sync_copy(
                acc.at[pl.ds(0, R * H)],
                out.at[pl.ds(pl.multiple_of(base * H, 8), R * H)])
            return carry

        lax.fori_loop(0, S_n, slab, 0)

    out = pl.kernel(body, out_type=out_type, mesh=mesh,
                    scratch_types=scratch,
                    compiler_params=_SC_PARAMS)(table, idxp, segp, starts,
                                                ends)
    return out.reshape(npad, H)


def _pad_entries(idx, seg):
    pad0 = jnp.zeros((SEG_PAD,), jnp.int32)
    padm = jnp.full((SEG_PAD,), -1, jnp.int32)
    return (jnp.concatenate([idx, pad0]), jnp.concatenate([seg, padm]))


def _chunk_bounds(seg_sorted, nchunk, CH):
    cb = jnp.searchsorted(
        seg_sorted, jnp.arange(nchunk + 1, dtype=jnp.int32) * CH
    ).astype(jnp.int32)
    starts = (cb[:-1] // 8) * 8
    ends = (cb[1:] // 8) * 8 + 8
    lpad = (nchunk // 8) * 8 + 16
    z = jnp.zeros((lpad - nchunk,), jnp.int32)
    return jnp.concatenate([starts, z]), jnp.concatenate([ends, z])


def _mlpa_body(x_ref, w1_ref, b1_ref, p1_ref, w2_ref, b2_ref, t1_ref, st_ref):
    i = pl.program_id(0)
    h = x_ref[...] @ w1_ref[...] + b1_ref[...]
    h = jnp.where(h >= 0, h, p1_ref[0, 0] * h)
    t = h @ w2_ref[...] + b2_ref[...]
    t1_ref[...] = t

    @pl.when(i == 0)
    def _():
        st_ref[...] = jnp.zeros_like(st_ref)

    st_ref[...] += _stat_rows(t)


def _mlpa(x, w1, b1, p1, w2, b2):
    return pl.pallas_call(
        _mlpa_body,
        grid=(N // BNR,),
        in_specs=[_rows((BNR, IN_F)), _full((IN_F, H)), _full((1, H)),
                  _full((1, 1)), _full((H, H)), _full((1, H))],
        out_specs=[_rows((BNR, H)), _full((8, H))],
        out_shape=[jax.ShapeDtypeStruct((N, H), _f32),
                   jax.ShapeDtypeStruct((8, H), _f32)],
    )(x, w1, b1.reshape(1, H), p1.reshape(1, 1), w2, b2.reshape(1, H))


def _mlpb_body(t1_ref, sc_ref, sh_ref, p2_ref, w3_ref, b3_ref, t2_ref, st_ref):
    i = pl.program_id(0)
    u = t1_ref[...] * sc_ref[...] + sh_ref[...]
    u = jnp.where(u >= 0, u, p2_ref[0, 0] * u)
    t = u @ w3_ref[...] + b3_ref[...]
    t2_ref[...] = t

    @pl.when(i == 0)
    def _():
        st_ref[...] = jnp.zeros_like(st_ref)

    st_ref[...] += _stat_rows(t)


def _mlpb(t1, sc, sh, p2, w3, b3):
    return pl.pallas_call(
        _mlpb_body,
        grid=(N // BNR,),
        in_specs=[_rows((BNR, H)), _full((1, H)), _full((1, H)),
                  _full((1, 1)), _full((H, H)), _full((1, H))],
        out_specs=[_rows((BNR, H)), _full((8, H))],
        out_shape=[jax.ShapeDtypeStruct((N, H), _f32),
                   jax.ShapeDtypeStruct((8, H), _f32)],
    )(t1, sc.reshape(1, H), sh.reshape(1, H), p2.reshape(1, 1), w3,
      b3.reshape(1, H))


def _mlpc_body(t2_ref, sc_ref, sh_ref, wi_ref, wj_ref, h_ref, hi_ref, hj_ref):
    hh = t2_ref[...] * sc_ref[...] + sh_ref[...]
    h_ref[...] = hh
    hi_ref[...] = hh @ wi_ref[...]
    hj_ref[...] = hh @ wj_ref[...]


def _mlpc(t2, sc, sh, wi, wj):
    return pl.pallas_call(
        _mlpc_body,
        grid=(N // BNR,),
        in_specs=[_rows((BNR, H)), _full((1, H)), _full((1, H)),
                  _full((H, H)), _full((H, H))],
        out_specs=[_rows((BNR, H))] * 3,
        out_shape=[jax.ShapeDtypeStruct((N, H), _f32)] * 3,
    )(t2, sc.reshape(1, H), sh.reshape(1, H), wi, wj)


def _edge_body(hid_ref, hjs_ref, hs_ref, ef_ref, bb_ref, sp_ref, sw_ref,
               sb_ref, eew_ref, eeb_ref, ea_ref, ew_ref):
    a = hid_ref[...] + hjs_ref[...] + bb_ref[...]
    a = jnp.where(a >= 0, a, sp_ref[0, 0] * a)
    t = a @ sw_ref[...] + sb_ref[...]
    ef = ef_ref[...] @ eew_ref[...] + eeb_ref[...]
    alpha = (t * ef).sum(-1, keepdims=True) * (1.0 / 16.0)
    ew = jax.nn.sigmoid(alpha)
    ew_ref[...] = ew
    ea_ref[...] = hs_ref[...] * ew


def _edge_gate(HiD, HjS, Hs, edge_feats, bb, sp, sw, sb, eew, eeb):
    return pl.pallas_call(
        _edge_body,
        grid=(E // BE,),
        in_specs=[_rows((BE, H)), _rows((BE, H)), _rows((BE, H)),
                  _rows((BE, ED)),
                  _full((1, H)), _full((1, 1)), _full((H, H)), _full((1, H)),
                  _full((ED, H)), _full((1, H))],
        out_specs=[_rows((BE, H)), _rows((BE, 1))],
        out_shape=[jax.ShapeDtypeStruct((E, H), _f32),
                   jax.ShapeDtypeStruct((E, 1), _f32)],
    )(HiD, HjS, Hs, edge_feats, bb.reshape(1, H), sp.reshape(1, 1), sw,
      sb.reshape(1, H), eew, eeb.reshape(1, H))


def _upd_body(ea_ref, agg_ref, ew_ref, out_ref):
    out_ref[...] = ea_ref[...] + agg_ref[...] * ew_ref[...]


def _lg_update(ea, agg, ew):
    return pl.pallas_call(
        _upd_body,
        grid=(E // BE,),
        in_specs=[_rows((BE, H)), _rows((BE, H)), _rows((BE, 1))],
        out_specs=_rows((BE, H)),
        out_shape=jax.ShapeDtypeStruct((E, H), _f32),
    )(ea, agg, ew)


def _f1_body(h_ref, ag_ref, h2_ref, st_ref):
    i = pl.program_id(0)
    t = h_ref[...] + ag_ref[...]
    h2_ref[...] = t

    @pl.when(i == 0)
    def _():
        st_ref[...] = jnp.zeros_like(st_ref)

    st_ref[...] += _stat_rows(t)


def _f1(h, aggh):
    return pl.pallas_call(
        _f1_body,
        grid=(N // BNR,),
        in_specs=[_rows((BNR, H)), _rows((BNR, H))],
        out_specs=[_rows((BNR, H)), _full((8, H))],
        out_shape=[jax.ShapeDtypeStruct((N, H), _f32),
                   jax.ShapeDtypeStruct((8, H), _f32)],
    )(h, aggh)


def _lin_body(zin_ref, sc_ref, sh_ref, p_ref, w_ref, wb_ref, z_ref, st_ref):
    i = pl.program_id(0)
    u = zin_ref[...] * sc_ref[...] + sh_ref[...]
    u = jnp.where(u >= 0, u, p_ref[0, 0] * u)
    t = u @ w_ref[...] + wb_ref[...]
    z_ref[...] = t

    @pl.when(i == 0)
    def _():
        st_ref[...] = jnp.zeros_like(st_ref)

    st_ref[...] += _stat_rows(t)


def _lin(zin, sc, sh, p, w, wb, fin, fout):
    return pl.pallas_call(
        _lin_body,
        grid=(N // BNR,),
        in_specs=[_rows((BNR, fin)), _full((1, fin)), _full((1, fin)),
                  _full((1, 1)), _full((fin, fout)), _full((1, fout))],
        out_specs=[_rows((BNR, fout)), _full((8, fout))],
        out_shape=[jax.ShapeDtypeStruct((N, fout), _f32),
                   jax.ShapeDtypeStruct((8, fout), _f32)],
    )(zin, sc.reshape(1, fin), sh.reshape(1, fin), p.reshape(1, 1), w,
      wb.reshape(1, fout))


def _mix_body(za_ref, zb_ref, z_ref, st_ref):
    i = pl.program_id(0)
    t = (za_ref[...] + zb_ref[...]) * 0.5
    z_ref[...] = t

    @pl.when(i == 0)
    def _():
        st_ref[...] = jnp.zeros_like(st_ref)

    st_ref[...] += _stat_rows(t)


def _mix(za, zb):
    return pl.pallas_call(
        _mix_body,
        grid=(N // BNR,),
        in_specs=[_rows((BNR, S)), _rows((BNR, S))],
        out_specs=[_rows((BNR, S)), _full((8, S))],
        out_shape=[jax.ShapeDtypeStruct((N, S), _f32),
                   jax.ShapeDtypeStruct((8, S), _f32)],
    )(za, zb)


def _pairproj_body(xj_ref, xi_ref, wk_ref, wq_ref, wip_ref, wjp_ref,
                   kj_ref, qi_ref, pi_ref, pj_ref):
    xj = xj_ref[...]
    xi = xi_ref[...]
    kj_ref[...] = xj @ wk_ref[...]
    qi_ref[...] = xi @ wq_ref[...]
    pi_ref[...] = xi @ wip_ref[...]
    pj_ref[...] = xj @ wjp_ref[...]


def _pairproj(xj, xi, wk, wq, wip, wjp):
    nj = xj.shape[0]
    return pl.pallas_call(
        _pairproj_body,
        grid=(nj // BJ,),
        in_specs=[_rows((BJ, S)), _rows((BJ, S))] + [_full((S, H))] * 4,
        out_specs=[_rows((BJ, H))] * 4,
        out_shape=[jax.ShapeDtypeStruct((nj, H), _f32)] * 4,
    )(xj, xi, wk, wq, wip, wjp)


def _pairatt_body(kj_ref, qi_ref, pi_ref, pj_ref, cb_ref, ca_ref, pv_ref):
    t = jnp.tanh(kj_ref[...] + qi_ref[...] + cb_ref[...])
    att = (t * ca_ref[...]).sum(-1, keepdims=True)
    pv_ref[...] = att * (pi_ref[...] * pj_ref[...])


def _pairatt(KjP, QiP, PiP, PjP, cb, ca):
    return pl.pallas_call(
        _pairatt_body,
        grid=(PE // BP,),
        in_specs=[_rows((BP, H))] * 4 + [_full((1, H)), _full((1, H))],
        out_specs=_rows((BP, H)),
        out_shape=jax.ShapeDtypeStruct((PE, H), _f32),
    )(KjP, QiP, PiP, PjP, cb.reshape(1, H), ca.reshape(1, H))


def _score_body(pg_ref, rg_ref, s_ref):
    s_ref[...] = (pg_ref[...] * rg_ref[...]).sum(-1, keepdims=True)


def _score(pair_g, r_g):
    n = pair_g.shape[0]
    return pl.pallas_call(
        _score_body,
        in_specs=[_full((n, H)), _full((n, H))],
        out_specs=_full((n, 1)),
        out_shape=jax.ShapeDtypeStruct((n, 1), _f32),
    )(pair_g, r_g)


def kernel(x, edge_feats, params, edge_index, line_graph_edge_index,
           pair_edge_index, edge_index_batch, rels, drug_pair_indices,
           node_j_for_pairs, node_i_for_pairs):
    p = params
    src, dst = edge_index[0], edge_index[1]

    t1, st1 = _mlpa(x, p['mlp_w1'], p['mlp_b1'], p['mlp_p1'], p['mlp_w2'],
                    p['mlp_b2'])
    sc1, sh1 = _bn_coeff(st1, N, p['mlp_bn1_g'], p['mlp_bn1_b'])
    t2, st2 = _mlpb(t1, sc1, sh1, p['mlp_p2'], p['mlp_w3'], p['mlp_b3'])
    sc2, sh2 = _bn_coeff(st2, N, p['mlp_bn2_g'], p['mlp_bn2_b'])
    h, hi, hj = _mlpc(t2, sc2, sh2, p['w_i'], p['w_j'])

    HiD, HjS, Hs = _sc_multigather([hi, hj, h], [dst, src], [0, 1, 1], 400)
    ea, ew = _edge_gate(HiD, HjS, Hs, edge_feats, p['blk_bias'], p['sml_p'],
                        p['sml_w'], p['sml_b'], p['ee_w'], p['ee_b'])

    lg1s, lg0s = lax.sort((line_graph_edge_index[1], line_graph_edge_index[0]),
                          num_keys=1)
    lg0p, lg1p = _pad_entries(lg0s, lg1s)
    lst, len_ = _chunk_bounds(lg1s, 800, 1000)
    out = ea
    for _ in range(N_ITER):
        agg = _sc_segsum(out, lg0p, lg1p, lst, len_, E, 1000)
        out = _lg_update(ea, agg, ew)

    dsts, eids = lax.sort((dst, jnp.arange(E, dtype=dst.dtype)), num_keys=1)
    eidp, dstp = _pad_entries(eids, dsts)
    nst, nen = _chunk_bounds(dsts, 64, 1000)
    aggh = _sc_segsum(out, eidp, dstp, nst, nen, 64000, 1000)[:N]

    h2, sth = _f1(h, aggh)
    scl1, shl1 = _bn_coeff(sth, N, p['l1_g'], p['l1_b'])
    z, stz = _lin(h2, scl1, shl1, jnp.float32(1.0), p['l1_w'], p['l1_wb'],
                  H, S)
    scz, shz = _bn_coeff(stz, N, p['l2_g'], p['l2_b'])
    z2, st22 = _lin(z, scz, shz, p['l2_p'], p['l2_w'], p['l2_wb'], S, S)
    sc3, sh3 = _bn_coeff(st22, N, p['l3_g'], p['l3_b'])
    z3, _ = _lin(z2, sc3, sh3, p['l3_p'], p['l3_w'], p['l3_wb'], S, S)
    zB, stB = _mix(z3, z)
    sc4, sh4 = _bn_coeff(stB, N, p['l4_g'], p['l4_b'])
    z4, _ = _lin(zB, sc4, sh4, p['l4_p'], p['l4_w'], p['l4_wb'], S, S)
    zC, _ = _mix(z4, zB)

    xj, xi = _sc_multigather([zC, zC], [node_j_for_pairs, node_i_for_pairs],
                             [0, 1], 256)
    Kj, Qi, Pi, Pj = _pairproj(xj, xi, p['ca_wk'], p['ca_wq'], p['i_pro'],
                               p['j_pro'])
    pe0, pe1 = pair_edge_index[0], pair_edge_index[1]
    KjP, QiP, PiP, PjP = _sc_multigather(
        [Kj, Qi, Pi, Pj], [pe0, pe1], [0, 1, 1, 0], 400)
    pv = _pairatt(KjP, QiP, PiP, PjP, p['ca_bias'], p['ca_a'])
    arp, eibp = _pad_entries(jnp.arange(PE, dtype=jnp.int32),
                             edge_index_batch)
    pst, pen = _chunk_bounds(edge_index_batch, 32, 64)
    seg = _sc_segsum(pv, arp, eibp, pst, pen, NPAIRS, 64)
    rels_full = jnp.concatenate([rels, rels])
    pair_g, r_g = _sc_multigather([seg, p['rel_embs']],
                                  [drug_pair_indices, rels_full],
                                  [0, 1], 64)
    scores = _score(pair_g, r_g)
    return scores[:B], scores[B:].reshape(B, 1, 1)
